# baseline (speedup 1.0000x reference)
import functools
import math

import jax
import jax.numpy as jnp
from jax import lax
from jax.experimental import pallas as pl
from jax.experimental.pallas import tpu as pltpu
from jax.experimental.pallas import tpu_sc as plsc

N_NODES = 1000000
N_DIM = 16
R = 10.0
BATCH = 16384

_NC = 2
_NS = 16
_NW = _NC * _NS
_E = 2 * BATCH
_EPW = _E // _NW
_G = _EPW // 16
_WIN = 2048
_TAIL = 640
_TB = N_NODES - _TAIL
_WMAX = (N_NODES - _WIN) // _WIN
_WT = _WMAX + 1


def _sc_gather(table_t, tail_t, sidx, spos):
    mesh = plsc.VectorSubcoreMesh(core_axis_name="c", subcore_axis_name="s")

    @functools.partial(
        pl.kernel,
        out_type=jax.ShapeDtypeStruct((_E * N_DIM,), jnp.float32),
        mesh=mesh,
        compiler_params=pltpu.CompilerParams(needs_layout_passes=False),
        scratch_types=[
            pltpu.VMEM((_EPW,), jnp.int32),
            pltpu.VMEM((_EPW,), jnp.int32),
            pltpu.VMEM((N_DIM, _WIN), jnp.float32),
            pltpu.VMEM((N_DIM, _WIN), jnp.float32),
            pltpu.VMEM((N_DIM, _TAIL), jnp.float32),
            pltpu.VMEM((_EPW * N_DIM,), jnp.float32),
            pltpu.SemaphoreType.DMA,
            pltpu.SemaphoreType.DMA,
            pltpu.SemaphoreType.DMA,
        ],
    )
    def k(tab_hbm, tail_hbm, sidx_hbm, spos_hbm, out_hbm,
          idx_v, pos_v, win_a, win_b, tail_v, cols_v, sem_a, sem_b, sem_o):
        wid = lax.axis_index("s") * _NC + lax.axis_index("c")
        base = wid * _EPW
        pltpu.sync_copy(sidx_hbm.at[pl.ds(base, _EPW)], idx_v)
        pltpu.sync_copy(spos_hbm.at[pl.ds(base, _EPW)], pos_v)
        pltpu.sync_copy(tail_hbm, tail_v)

        lanes = lax.iota(jnp.int32, 16)

        def fetch_sync(w):
            ws = pl.multiple_of(w * _WIN, 128)

            @pl.when(lax.rem(w, 2) == 0)
            def _():
                pltpu.sync_copy(tab_hbm.at[:, pl.ds(ws, _WIN)], win_a)

            @pl.when(lax.rem(w, 2) == 1)
            def _():
                pltpu.sync_copy(tab_hbm.at[:, pl.ds(ws, _WIN)], win_b)

        def fetch_async(w):
            ws = pl.multiple_of(w * _WIN, 128)

            @pl.when(lax.rem(w, 2) == 0)
            def _():
                pltpu.async_copy(
                    tab_hbm.at[:, pl.ds(ws, _WIN)], win_a, sem_a)

            @pl.when(lax.rem(w, 2) == 1)
            def _():
                pltpu.async_copy(
                    tab_hbm.at[:, pl.ds(ws, _WIN)], win_b, sem_b)

        def wait_win(w):
            @pl.when(lax.rem(w, 2) == 0)
            def _():
                pltpu.make_async_copy(
                    tab_hbm.at[:, pl.ds(0, _WIN)], win_a, sem_a).wait()

            @pl.when(lax.rem(w, 2) == 1)
            def _():
                pltpu.make_async_copy(
                    tab_hbm.at[:, pl.ds(0, _WIN)], win_b, sem_b).wait()

        w0 = jnp.minimum(idx_v[pl.ds(0, 16)][0] // _WIN, _WMAX)
        fetch_sync(w0)
        pf0 = jnp.minimum(w0 + 1, _WMAX)
        fetch_async(pf0)

        def group(g, carry):
            iv0 = idx_v[pl.ds(g * 16, 16)]
            wv0 = jnp.where(iv0 >= _TB, _WT, iv0 // _WIN)
            wlo = jnp.min(wv0)
            whi = jnp.max(wv0)

            def win_iter(w, carry2):
                cur, pf = carry2
                trans = (w <= _WMAX) & (w != cur)

                @pl.when(trans)
                def _():
                    wait_win(pf)

                @pl.when(trans & (w != pf))
                def _():
                    fetch_sync(w)

                pfid = jnp.minimum(w + 1, _WMAX)

                @pl.when(trans)
                def _():
                    fetch_async(pfid)

                cur = lax.select(trans, w, cur)
                pf = lax.select(trans, pfid, pf)
                even = lax.rem(w, 2) == 0
                is_tail_w = w == _WT
                m16 = (wv0 == w).astype(jnp.int32)

                for l in range(16):
                    r = iv0[l]
                    cw = jnp.full(
                        (16,),
                        jnp.clip(r - w * _WIN, 0, _WIN - 1), jnp.int32)
                    ct = jnp.full(
                        (16,),
                        jnp.clip(r - _TB, 0, _TAIL - 1), jnp.int32)
                    col_a = plsc.load_gather(win_a, [lanes, cw])
                    col_b = plsc.load_gather(win_b, [lanes, cw])
                    col_t = plsc.load_gather(tail_v, [lanes, ct])
                    col = jnp.where(
                        is_tail_w, col_t,
                        jnp.where(even, col_a, col_b))
                    plsc.store_scatter(
                        cols_v, [(g * 16 + l) * N_DIM + lanes], col,
                        mask=jnp.full((16,), m16[l]) != 0)

                return (cur, pf)

            return lax.fori_loop(wlo, whi + 1, win_iter, carry)

        _, pf_end = lax.fori_loop(0, _G, group, (w0, pf0))
        wait_win(pf_end)

        def outs(g, _):
            ip = pos_v[pl.ds(g * 16, 16)]
            for l in range(16):
                j = g * 16 + l
                pltpu.async_copy(
                    cols_v.at[pl.ds(j * N_DIM, N_DIM)],
                    out_hbm.at[pl.ds(ip[l] * N_DIM, N_DIM)], sem_o)
            return ()

        lax.fori_loop(0, _G, outs, ())

        def drain_out(i, _):
            pltpu.make_async_copy(
                cols_v.at[pl.ds(0, N_DIM)],
                out_hbm.at[pl.ds(0, N_DIM)], sem_o).wait()
            return ()

        lax.fori_loop(0, _EPW, drain_out, ())

    return k(table_t, tail_t, sidx, spos)


def _tc_loss(rows1d, labels2d, beta):
    const = N_DIM * math.log(2.0 * math.pi)
    inv = 1.0 / (N_NODES - 1)
    blk = 2048
    nblk = BATCH // blk
    rows = blk * N_DIM // 128

    def body(beta_ref, u_ref, v_ref, y_ref, o_ref):
        u = u_ref[...].reshape(rows, 128)
        v = v_ref[...].reshape(rows, 128)
        bd = (lax.broadcasted_iota(jnp.int32, (128, 8), 0) // N_DIM
              == lax.broadcasted_iota(jnp.int32, (128, 8), 1)
              ).astype(jnp.float32)
        du = u - v
        d2 = jnp.dot(du * du, bd, preferred_element_type=jnp.float32)
        t = jnp.dot(u * u + v * v, bd, preferred_element_type=jnp.float32)
        dist = jnp.sqrt(d2 + 1e-12)
        z = beta_ref[0] * (dist - R)
        y = y_ref[...].astype(jnp.float32)
        loss = y * jnp.logaddexp(0.0, z) + (1.0 - y) * jnp.logaddexp(0.0, -z)
        o_ref[...] = loss + (const + 0.5 * t) * inv

    return pl.pallas_call(
        body,
        grid=(nblk,),
        in_specs=[
            pl.BlockSpec(memory_space=pltpu.SMEM),
            pl.BlockSpec((blk * N_DIM,), lambda i: (i,)),
            pl.BlockSpec((blk * N_DIM,), lambda i: (i + nblk,)),
            pl.BlockSpec((rows, 8), lambda i: (i, 0)),
        ],
        out_specs=pl.BlockSpec((rows, 8), lambda i: (i, 0)),
        out_shape=jax.ShapeDtypeStruct((BATCH // 8, 8), jnp.float32),
    )(jnp.reshape(beta, (1,)).astype(jnp.float32), rows1d, rows1d, labels2d)


def kernel(pairs, labels, table, beta):
    table_t = table.T
    tail_t = table_t[:, _TB:]
    idx_flat = pairs.T.reshape(-1)
    pos = lax.iota(jnp.int32, _E)
    sidx, spos = lax.sort_key_val(idx_flat, pos)
    rows1d = _sc_gather(table_t, tail_t, sidx, spos)
    loss2d = _tc_loss(rows1d, labels.reshape(BATCH // 8, 8), beta)
    return loss2d.reshape(BATCH)


# speedup vs baseline: 1.0349x; 1.0349x over previous
import functools
import math

import jax
import jax.numpy as jnp
from jax import lax
from jax.experimental import pallas as pl
from jax.experimental.pallas import tpu as pltpu
from jax.experimental.pallas import tpu_sc as plsc

N_NODES = 1000000
N_DIM = 16
R = 10.0
BATCH = 16384

_NC = 2
_NS = 16
_NW = _NC * _NS
_E = 2 * BATCH
_EPW = _E // _NW
_G = _EPW // 16
_WIN = 2048
_TAIL = 640
_TB = N_NODES - _TAIL
_WMAX = (N_NODES - _WIN) // _WIN
_WT = _WMAX + 1


def _sc_gather(table_t, tail_t, sidx, spos):
    mesh = plsc.VectorSubcoreMesh(core_axis_name="c", subcore_axis_name="s")

    @functools.partial(
        pl.kernel,
        out_type=jax.ShapeDtypeStruct((_E * N_DIM,), jnp.float32),
        mesh=mesh,
        compiler_params=pltpu.CompilerParams(needs_layout_passes=False),
        scratch_types=[
            pltpu.VMEM((_EPW,), jnp.int32),
            pltpu.VMEM((_EPW,), jnp.int32),
            pltpu.VMEM((N_DIM, _WIN), jnp.float32),
            pltpu.VMEM((N_DIM, _WIN), jnp.float32),
            pltpu.VMEM((N_DIM, _TAIL), jnp.float32),
            pltpu.VMEM((_EPW * N_DIM,), jnp.float32),
            pltpu.SemaphoreType.DMA,
            pltpu.SemaphoreType.DMA,
            pltpu.SemaphoreType.DMA,
        ],
    )
    def k(tab_hbm, tail_hbm, sidx_hbm, spos_hbm, out_hbm,
          idx_v, pos_v, win_a, win_b, tail_v, cols_v, sem_a, sem_b, sem_o):
        wid = lax.axis_index("s") * _NC + lax.axis_index("c")
        base = wid * _EPW
        pltpu.sync_copy(sidx_hbm.at[pl.ds(base, _EPW)], idx_v)
        pltpu.sync_copy(spos_hbm.at[pl.ds(base, _EPW)], pos_v)
        pltpu.sync_copy(tail_hbm, tail_v)

        lanes = lax.iota(jnp.int32, 16)

        def fetch_sync(w):
            ws = pl.multiple_of(w * _WIN, 128)

            @pl.when(lax.rem(w, 2) == 0)
            def _():
                pltpu.sync_copy(tab_hbm.at[:, pl.ds(ws, _WIN)], win_a)

            @pl.when(lax.rem(w, 2) == 1)
            def _():
                pltpu.sync_copy(tab_hbm.at[:, pl.ds(ws, _WIN)], win_b)

        def fetch_async(w):
            ws = pl.multiple_of(w * _WIN, 128)

            @pl.when(lax.rem(w, 2) == 0)
            def _():
                pltpu.async_copy(
                    tab_hbm.at[:, pl.ds(ws, _WIN)], win_a, sem_a)

            @pl.when(lax.rem(w, 2) == 1)
            def _():
                pltpu.async_copy(
                    tab_hbm.at[:, pl.ds(ws, _WIN)], win_b, sem_b)

        def wait_win(w):
            @pl.when(lax.rem(w, 2) == 0)
            def _():
                pltpu.make_async_copy(
                    tab_hbm.at[:, pl.ds(0, _WIN)], win_a, sem_a).wait()

            @pl.when(lax.rem(w, 2) == 1)
            def _():
                pltpu.make_async_copy(
                    tab_hbm.at[:, pl.ds(0, _WIN)], win_b, sem_b).wait()

        w0 = jnp.minimum(idx_v[pl.ds(0, 16)][0] // _WIN, _WMAX)
        fetch_sync(w0)
        pf0 = jnp.minimum(w0 + 1, _WMAX)
        fetch_async(pf0)

        def group(g, carry):
            iv0 = idx_v[pl.ds(g * 16, 16)]
            ip = pos_v[pl.ds(g * 16, 16)]
            wv0 = jnp.where(iv0 >= _TB, _WT, iv0 // _WIN)
            wlo = jnp.min(wv0)
            whi = jnp.max(wv0)

            def win_iter(w, carry2):
                cur, pf = carry2
                trans = (w <= _WMAX) & (w != cur)

                @pl.when(trans)
                def _():
                    wait_win(pf)

                @pl.when(trans & (w != pf))
                def _():
                    fetch_sync(w)

                pfid = jnp.minimum(w + 1, _WMAX)

                @pl.when(trans)
                def _():
                    fetch_async(pfid)

                cur = lax.select(trans, w, cur)
                pf = lax.select(trans, pfid, pf)
                even = lax.rem(w, 2) == 0
                is_tail_w = w == _WT

                for l in range(16):
                    r = iv0[l]

                    @pl.when(wv0[l] == w)
                    def _():
                        cw = jnp.full(
                            (16,),
                            jnp.clip(r - w * _WIN, 0, _WIN - 1), jnp.int32)
                        ct = jnp.full(
                            (16,),
                            jnp.clip(r - _TB, 0, _TAIL - 1), jnp.int32)
                        col_a = plsc.load_gather(win_a, [lanes, cw])
                        col_b = plsc.load_gather(win_b, [lanes, cw])
                        col_t = plsc.load_gather(tail_v, [lanes, ct])
                        col = jnp.where(
                            is_tail_w, col_t,
                            jnp.where(even, col_a, col_b))
                        j = g * 16 + l
                        cols_v[pl.ds(j * N_DIM, N_DIM)] = col
                        pltpu.async_copy(
                            cols_v.at[pl.ds(j * N_DIM, N_DIM)],
                            out_hbm.at[pl.ds(ip[l] * N_DIM, N_DIM)], sem_o)

                return (cur, pf)

            return lax.fori_loop(wlo, whi + 1, win_iter, carry)

        _, pf_end = lax.fori_loop(0, _G, group, (w0, pf0))
        wait_win(pf_end)

        def drain_out(i, _):
            pltpu.make_async_copy(
                cols_v.at[pl.ds(0, N_DIM)],
                out_hbm.at[pl.ds(0, N_DIM)], sem_o).wait()
            return ()

        lax.fori_loop(0, _EPW, drain_out, ())

    return k(table_t, tail_t, sidx, spos)


def _tc_loss(rows1d, labels2d, beta):
    const = N_DIM * math.log(2.0 * math.pi)
    inv = 1.0 / (N_NODES - 1)
    blk = 2048
    nblk = BATCH // blk
    rows = blk * N_DIM // 128

    def body(beta_ref, u_ref, v_ref, y_ref, o_ref):
        u = u_ref[...].reshape(rows, 128)
        v = v_ref[...].reshape(rows, 128)
        bd = (lax.broadcasted_iota(jnp.int32, (128, 8), 0) // N_DIM
              == lax.broadcasted_iota(jnp.int32, (128, 8), 1)
              ).astype(jnp.float32)
        du = u - v
        d2 = jnp.dot(du * du, bd, preferred_element_type=jnp.float32)
        t = jnp.dot(u * u + v * v, bd, preferred_element_type=jnp.float32)
        dist = jnp.sqrt(d2 + 1e-12)
        z = beta_ref[0] * (dist - R)
        y = y_ref[...].astype(jnp.float32)
        loss = y * jnp.logaddexp(0.0, z) + (1.0 - y) * jnp.logaddexp(0.0, -z)
        o_ref[...] = loss + (const + 0.5 * t) * inv

    return pl.pallas_call(
        body,
        grid=(nblk,),
        in_specs=[
            pl.BlockSpec(memory_space=pltpu.SMEM),
            pl.BlockSpec((blk * N_DIM,), lambda i: (i,)),
            pl.BlockSpec((blk * N_DIM,), lambda i: (i + nblk,)),
            pl.BlockSpec((rows, 8), lambda i: (i, 0)),
        ],
        out_specs=pl.BlockSpec((rows, 8), lambda i: (i, 0)),
        out_shape=jax.ShapeDtypeStruct((BATCH // 8, 8), jnp.float32),
    )(jnp.reshape(beta, (1,)).astype(jnp.float32), rows1d, rows1d, labels2d)


def kernel(pairs, labels, table, beta):
    table_t = table.T
    tail_t = table_t[:, _TB:]
    idx_flat = pairs.T.reshape(-1)
    pos = lax.iota(jnp.int32, _E)
    sidx, spos = lax.sort_key_val(idx_flat, pos)
    rows1d = _sc_gather(table_t, tail_t, sidx, spos)
    loss2d = _tc_loss(rows1d, labels.reshape(BATCH // 8, 8), beta)
    return loss2d.reshape(BATCH)
